# Initial kernel scaffold; baseline (speedup 1.0000x reference)
import jax
import jax.numpy as jnp
from jax.experimental import pallas as pl


def kernel(x, edge_index, edge_attr, batch, node_type, emb, conv_w, conv_b, fuse_w, fuse_b, e1_w1, e1_b1, e1_w2, e1_b2, n1_w1, n1_b1, n1_w2, n1_b2, e2_w1, e2_b1, e2_w2, e2_b2, n2_w1, n2_b1, n2_w2, n2_b2, f_w1, f_b1, bn_g, bn_b, f_w2, f_b2):
    raise NotImplementedError("write your pallas kernel here")


# speedup vs baseline: 2.5638x; 2.5638x over previous
import functools

import jax
import jax.numpy as jnp
import numpy as np
from jax import lax
from jax.experimental import pallas as pl
from jax.experimental.pallas import tpu as pltpu
from jax.experimental.pallas import tpu_sc as plsc

N = 10000
E = 160000
NP = 10240
BE = 1280
BN = 2000
CH = 128
NW = 32

_f32 = jnp.float32


def _node_enc_body(xn, w21, bh, a1, b1w, n1a, p1, p2, p3):
    h = jnp.dot(xn[...], w21[...], preferred_element_type=_f32) + bh[...]
    p1[...] = jnp.dot(h, a1[...], preferred_element_type=_f32)
    p2[...] = jnp.dot(h, b1w[...], preferred_element_type=_f32)
    p3[...] = jnp.dot(h, n1a[...], preferred_element_type=_f32)


def _edge1_body(gr, gc, ea, wc, b1, w2, b2, o):
    u = gr[...] + gc[...] + jnp.dot(ea[...], wc[...], preferred_element_type=_f32) + b1[...]
    o[...] = jnp.dot(jax.nn.relu(u), w2[...], preferred_element_type=_f32) + b2[...]


def _node1_body(g3, ea1n, n1b, n1b1, n1w2, n1b2, e2a, e2b, q1, q2):
    u = jax.nn.relu(g3[...] + jnp.dot(ea1n[...], n1b[...], preferred_element_type=_f32) + n1b1[...])
    h1 = jnp.dot(u, n1w2[...], preferred_element_type=_f32) + n1b2[...]
    q1[...] = jnp.dot(h1, e2a[...], preferred_element_type=_f32)
    q2[...] = jnp.dot(h1, e2b[...], preferred_element_type=_f32)


def _edge2_body(gr2, gc2, ea1, e2c, e2b1, e2w2, e2b2, fw1, fb1, z_ref, st_ref):
    u = jax.nn.relu(gr2[...] + gc2[...] +
                    jnp.dot(ea1[...], e2c[...], preferred_element_type=_f32) + e2b1[...])
    v = jnp.dot(u, e2w2[...], preferred_element_type=_f32) + e2b2[...]
    z = jnp.dot(v, fw1[...], preferred_element_type=_f32) + fb1[...]
    z_ref[...] = z
    s = jnp.sum(z, axis=0).reshape(1, 128)
    sq = jnp.sum(z * z, axis=0).reshape(1, 128)
    upd = jnp.concatenate([s, sq, jnp.zeros((6, 128), _f32)], axis=0)

    @pl.when(pl.program_id(0) == 0)
    def _():
        st_ref[...] = jnp.zeros((8, 128), _f32)

    st_ref[...] += upd


def _final_body(z, av, cv, fw2, fb2, o):
    u = jax.nn.relu(z[...] * av[...] + cv[...])
    o[...] = jnp.dot(u, fw2[...], preferred_element_type=_f32) + fb2[...]


def _w_spec(shape):
    return pl.BlockSpec(shape, lambda i: (0,) * len(shape))


def _run_node_enc(Xn, W21, bh, a1, b1w, n1a):
    grid = (N // BN,)
    blk = lambda r, c: pl.BlockSpec((r, c), lambda i: (i, 0))
    return pl.pallas_call(
        _node_enc_body,
        grid=grid,
        in_specs=[blk(BN, 21), _w_spec((21, 128)), _w_spec((1, 128)),
                  _w_spec((128, 128)), _w_spec((128, 128)), _w_spec((128, 128))],
        out_specs=[blk(BN, 128)] * 3,
        out_shape=[jax.ShapeDtypeStruct((N, 128), _f32)] * 3,
    )(Xn, W21, bh, a1, b1w, n1a)


def _run_edge1(gr, gc, ea, wc, b1, w2, b2):
    grid = (E // BE,)
    blk = lambda r, c: pl.BlockSpec((r, c), lambda i: (i, 0))
    return pl.pallas_call(
        _edge1_body,
        grid=grid,
        in_specs=[blk(BE, 128), blk(BE, 128), blk(BE, 4), _w_spec((4, 128)),
                  _w_spec((1, 128)), _w_spec((128, 128)), _w_spec((1, 128))],
        out_specs=blk(BE, 128),
        out_shape=jax.ShapeDtypeStruct((E, 128), _f32),
    )(gr, gc, ea, wc, b1, w2, b2)


def _run_node1(g3p, ea1, n1b, n1b1, n1w2, n1b2, e2a, e2b):
    grid = (N // BN,)
    blk = lambda r, c: pl.BlockSpec((r, c), lambda i: (i, 0))
    return pl.pallas_call(
        _node1_body,
        grid=grid,
        in_specs=[blk(BN, 128), blk(BN, 128), _w_spec((128, 128)), _w_spec((1, 128)),
                  _w_spec((128, 128)), _w_spec((1, 128)),
                  _w_spec((128, 128)), _w_spec((128, 128))],
        out_specs=[blk(BN, 128)] * 2,
        out_shape=[jax.ShapeDtypeStruct((N, 128), _f32)] * 2,
    )(g3p, ea1, n1b, n1b1, n1w2, n1b2, e2a, e2b)


def _run_edge2(gr2, gc2, ea1, e2c, e2b1, e2w2, e2b2, fw1, fb1):
    grid = (E // BE,)
    blk = lambda r, c: pl.BlockSpec((r, c), lambda i: (i, 0))
    return pl.pallas_call(
        _edge2_body,
        grid=grid,
        in_specs=[blk(BE, 128), blk(BE, 128), blk(BE, 128), _w_spec((128, 128)),
                  _w_spec((1, 128)), _w_spec((128, 128)), _w_spec((1, 128)),
                  _w_spec((128, 128)), _w_spec((1, 128))],
        out_specs=[blk(BE, 128), pl.BlockSpec((8, 128), lambda i: (0, 0))],
        out_shape=[jax.ShapeDtypeStruct((E, 128), _f32),
                   jax.ShapeDtypeStruct((8, 128), _f32)],
    )(gr2, gc2, ea1, e2c, e2b1, e2w2, e2b2, fw1, fb1)


def _run_final(z, av, cv, fw2, fb2):
    grid = (E // BE,)
    blk = lambda r, c: pl.BlockSpec((r, c), lambda i: (i, 0))
    return pl.pallas_call(
        _final_body,
        grid=grid,
        in_specs=[blk(BE, 128), _w_spec((1, 128)), _w_spec((1, 128)),
                  _w_spec((128, 3)), _w_spec((1, 3))],
        out_specs=blk(BE, 3),
        out_shape=jax.ShapeDtypeStruct((E, 3), _f32),
    )(z, av, cv, fw2, fb2)


def _sc_mesh():
    return plsc.VectorSubcoreMesh(core_axis_name="c", subcore_axis_name="s")


def _gather_chunks(wid, tbl, idx_hbm, out_hbm, nchunks, idx_v, rows_v, sem):
    iters = (nchunks + NW - 1) // NW

    @pl.loop(0, iters)
    def _(i):
        chunk = i * NW + wid

        @pl.when(chunk < nchunks)
        def _():
            base = chunk * CH
            pltpu.sync_copy(idx_hbm.at[pl.ds(base, CH)], idx_v)
            pltpu.async_copy(tbl.at[idx_v], rows_v, sem).wait()
            pltpu.sync_copy(rows_v, out_hbm.at[pl.ds(base, CH)])


def _sc_gather3(p1, p2, p3, rowE, colE, rowN):
    @functools.partial(
        pl.kernel,
        mesh=_sc_mesh(),
        out_type=(jax.ShapeDtypeStruct((E, 128), _f32),
                  jax.ShapeDtypeStruct((E, 128), _f32),
                  jax.ShapeDtypeStruct((NP, 128), _f32)),
        scratch_types=[pltpu.VMEM((CH,), jnp.int32),
                       pltpu.VMEM((CH, 128), _f32),
                       pltpu.SemaphoreType.DMA],
    )
    def k(p1_h, p2_h, p3_h, row_h, col_h, rown_h, gr_h, gc_h, g3_h,
          idx_v, rows_v, sem):
        wid = lax.axis_index("s") * 2 + lax.axis_index("c")
        _gather_chunks(wid, p1_h, row_h, gr_h, E // CH, idx_v, rows_v, sem)
        _gather_chunks(wid, p2_h, col_h, gc_h, E // CH, idx_v, rows_v, sem)
        _gather_chunks(wid, p3_h, rown_h, g3_h, NP // CH, idx_v, rows_v, sem)

    return k(p1, p2, p3, rowE, colE, rowN)


def _sc_gather2(q1, q2, rowE, colE):
    @functools.partial(
        pl.kernel,
        mesh=_sc_mesh(),
        out_type=(jax.ShapeDtypeStruct((E, 128), _f32),
                  jax.ShapeDtypeStruct((E, 128), _f32)),
        scratch_types=[pltpu.VMEM((CH,), jnp.int32),
                       pltpu.VMEM((CH, 128), _f32),
                       pltpu.SemaphoreType.DMA],
    )
    def k(q1_h, q2_h, row_h, col_h, gr_h, gc_h, idx_v, rows_v, sem):
        wid = lax.axis_index("s") * 2 + lax.axis_index("c")
        _gather_chunks(wid, q1_h, row_h, gr_h, E // CH, idx_v, rows_v, sem)
        _gather_chunks(wid, q2_h, col_h, gc_h, E // CH, idx_v, rows_v, sem)

    return k(q1, q2, rowE, colE)


def kernel(x, edge_index, edge_attr, batch, node_type, emb, conv_w, conv_b,
           fuse_w, fuse_b, e1_w1, e1_b1, e1_w2, e1_b2, n1_w1, n1_b1, n1_w2,
           n1_b2, e2_w1, e2_b1, e2_w2, e2_b2, n2_w1, n2_b1, n2_w2, n2_b2,
           f_w1, f_b1, bn_g, bn_b, f_w2, f_b2):
    L = 5
    w0 = conv_w[:, :, 0]; w1 = conv_w[:, :, 1]; w2 = conv_w[:, :, 2]
    ws = w0 + w1 + w2
    M = jnp.concatenate([ws[:, :4].T, -w2[:, :4].T, -w0[:, :4].T], axis=0) / L
    s_pe = emb.sum(0)
    const = (s_pe @ ws[:, 4:].T - emb[4] @ w0[:, 4:].T - emb[0] @ w2[:, 4:].T) / L + conv_b
    S = np.zeros((20, 12), np.float32)
    for i in range(4):
        for l in range(5):
            S[l * 4 + i, i] = 1.0
        S[0 * 4 + i, 4 + i] = 1.0
        S[4 * 4 + i, 8 + i] = 1.0
    W_x = (jnp.asarray(S) @ M) @ fuse_w[:128]
    b_h = (const @ fuse_w[:128] + fuse_b).reshape(1, 128)
    W21 = jnp.concatenate([W_x, fuse_w[128:129]], axis=0)
    Xn = jnp.concatenate([x.reshape(N, 20), node_type], axis=1)

    rowE = edge_index[0]
    colE = edge_index[1]
    rowN = jnp.concatenate([rowE[:N], jnp.zeros((NP - N,), jnp.int32)])

    r1 = lambda v: v.reshape(1, -1)

    p1, p2, p3 = _run_node_enc(Xn, W21, b_h, e1_w1[:128], e1_w1[128:256],
                               n1_w1[:128])
    gr, gc, g3p = _sc_gather3(p1, p2, p3, rowE, colE, rowN)
    ea1 = _run_edge1(gr, gc, edge_attr, e1_w1[256:260], r1(e1_b1),
                     e1_w2, r1(e1_b2))
    q1, q2 = _run_node1(g3p, ea1, n1_w1[128:256], r1(n1_b1), n1_w2, r1(n1_b2),
                        e2_w1[:128], e2_w1[128:256])
    gr2, gc2 = _sc_gather2(q1, q2, rowE, colE)
    z, st = _run_edge2(gr2, gc2, ea1, e2_w1[256:384], r1(e2_b1), e2_w2,
                       r1(e2_b2), f_w1, r1(f_b1))
    mu = st[0] / E
    var = st[1] / E - mu * mu
    a = bn_g / jnp.sqrt(var + 1e-5)
    c = bn_b - mu * a
    return _run_final(z, r1(a), r1(c), f_w2, r1(f_b2))
